# baseline (speedup 1.0000x reference)
import functools

import jax
import jax.numpy as jnp
from jax import lax
from jax.experimental import pallas as pl
from jax.experimental.pallas import tpu as pltpu
from jax.experimental.pallas import tpu_sc as plsc

N = 10000
NPAD = 10112
RPT = 632
D = 256
DC = 128
E = 160000
NC = 2
NS = 16
B = 128
EPAD = 163840
NBTOT = EPAD // B
NB = 80

CHUNK_T0 = [0, 2, 4, 7]
CHUNK_NS = [2, 2, 3, 3]
TMAP = [0, 1, 0, 1, 0, 1, 2, 0, 1, 2]


def _mm_body(x_ref, w_ref, o_ref):
    o_ref[0] = jnp.dot(x_ref[...], w_ref[0], preferred_element_type=jnp.float32)


def _make_tables(x, wstack):
    BN = 2000
    return pl.pallas_call(
        _mm_body,
        grid=(N // BN, 10),
        in_specs=[
            pl.BlockSpec((BN, D), lambda i, t: (i, 0)),
            pl.BlockSpec((1, D, DC), lambda i, t: (t, 0, 0)),
        ],
        out_specs=pl.BlockSpec((1, BN, DC), lambda i, t: (t, i, 0)),
        out_shape=jax.ShapeDtypeStruct((10, N, DC), jnp.float32),
    )(x, wstack)


def _sc_body(tabs, srcw, dstw, valw, zer,
             out, acc, sb3, db3, vb3, rows2, gsem, ssem, esem):
    cid = lax.axis_index("c")
    sid = lax.axis_index("s")
    nb = NB
    ebase = sid * NB * B

    @pl.when(cid == 0)
    def _sc0_work():
      for c in range(4):
        pltpu.sync_copy(zer, acc.at[pl.ds(sid * RPT, RPT), :])
        plsc.subcore_barrier()

        def _subtask(j, _, t0=CHUNK_T0[c]):
            tb = (t0 + j) * EPAD + ebase

            def _esmall(b, s3):
                off = tb + b * B
                pltpu.async_copy(srcw.at[pl.ds(off, B)], sb3.at[s3],
                                 esem.at[s3])
                pltpu.async_copy(dstw.at[pl.ds(off, B)], db3.at[s3],
                                 esem.at[s3])
                pltpu.async_copy(valw.at[pl.ds(off, B)], vb3.at[s3],
                                 esem.at[s3])

            def _ewait(b, s3):
                off = tb + b * B
                pltpu.make_async_copy(srcw.at[pl.ds(off, B)], sb3.at[s3],
                                      esem.at[s3]).wait()
                pltpu.make_async_copy(dstw.at[pl.ds(off, B)], db3.at[s3],
                                      esem.at[s3]).wait()
                pltpu.make_async_copy(valw.at[pl.ds(off, B)], vb3.at[s3],
                                      esem.at[s3]).wait()

            def _gather(s3, bi):
                pltpu.async_copy(tabs.at[sb3.at[s3]], rows2.at[bi],
                                 gsem.at[bi])

            def _gwait(s3, bi):
                pltpu.make_async_copy(tabs.at[sb3.at[s3]], rows2.at[bi],
                                      gsem.at[bi]).wait()

            def _swait(s3, bi):
                pltpu.make_async_copy(rows2.at[bi], acc.at[db3.at[s3]],
                                      ssem.at[bi]).wait()

            _esmall(0, 0)
            _ewait(0, 0)
            _gather(0, 0)
            _esmall(1, 1)

            def _batch(b, _):
                bi = b % 2
                bo = (b + 1) % 2
                s3 = b % 3

                @pl.when(b + 1 < nb)
                def _():
                    @pl.when(b >= 1)
                    def _():
                        _swait((b - 1) % 3, bo)
                    _ewait(b + 1, (b + 1) % 3)
                    _gather((b + 1) % 3, bo)

                _gwait(s3, bi)

                @plsc.parallel_loop(0, 8, unroll=2)
                def _scale(g):
                    vv = vb3[s3, pl.ds(g * 16, 16)]
                    for l in range(16):
                        v = vv[l]
                        e = g * 16 + l
                        loads = [rows2[bi, e, pl.ds(jj * 16, 16)]
                                 for jj in range(8)]
                        for jj in range(8):
                            rows2[bi, e, pl.ds(jj * 16, 16)] = loads[jj] * v

                pltpu.async_copy(rows2.at[bi], acc.at[db3.at[s3]],
                                 ssem.at[bi], add=True)

                @pl.when(b + 2 < nb)
                def _():
                    _esmall(b + 2, (b + 2) % 3)
                return 0

            lax.fori_loop(0, nb, _batch, 0)
            _swait((nb - 2) % 3, nb % 2)
            _swait((nb - 1) % 3, (nb - 1) % 2)
            return 0

        lax.fori_loop(0, CHUNK_NS[c], _subtask, 0)

        plsc.subcore_barrier()
        pltpu.sync_copy(acc.at[pl.ds(sid * RPT, RPT), :],
                        out.at[c, pl.ds(sid * RPT, RPT), :])


def _sc_spmm(tabs, srcw, dstw, valw, zer):
    mesh = plsc.VectorSubcoreMesh(core_axis_name="c", subcore_axis_name="s")
    kern = functools.partial(
        pl.kernel,
        mesh=mesh,
        out_type=jax.ShapeDtypeStruct((4, NPAD, DC), jnp.float32),
        scratch_types=[
            pltpu.VMEM_SHARED((NPAD, DC), jnp.float32),
            pltpu.VMEM((3, B), jnp.int32),
            pltpu.VMEM((3, B), jnp.int32),
            pltpu.VMEM((3, B), jnp.float32),
            pltpu.VMEM((2, B, DC), jnp.float32),
            pltpu.SemaphoreType.DMA((2,)),
            pltpu.SemaphoreType.DMA((2,)),
            pltpu.SemaphoreType.DMA((3,)),
        ],
    )(_sc_body)
    return kern(tabs, srcw, dstw, valw, zer)


def _ep_body(p_ref, x_ref, o_ref):
    c0 = p_ref[0]
    c1 = p_ref[1]
    c2 = p_ref[2]
    c3 = p_ref[3]
    o_ref[:, 0:DC] = jnp.maximum(jnp.maximum(c0, c2), 0.0)
    o_ref[:, DC:2 * DC] = jnp.maximum(jnp.maximum(c1, c3), 0.0)
    o_ref[:, 2 * DC:] = x_ref[...]


def _epilogue(part, x):
    BN = 2000
    return pl.pallas_call(
        _ep_body,
        grid=(N // BN,),
        in_specs=[
            pl.BlockSpec((4, BN, DC), lambda i: (0, i, 0)),
            pl.BlockSpec((BN, D), lambda i: (i, 0)),
        ],
        out_specs=pl.BlockSpec((BN, 2 * D), lambda i: (i, 0)),
        out_shape=jax.ShapeDtypeStruct((N, 2 * D), jnp.float32),
    )(part, x)


def kernel(x, s0_idx, s0_val, s1_idx, s1_val, s2_idx, s2_val,
           W00, W01, W10, W11, W12):
    x = x.astype(jnp.float32)
    wstack = jnp.stack([
        W00[:, :DC], W01[:, :DC],
        W00[:, DC:], W01[:, DC:],
        W10[:, :DC], W11[:, :DC], W12[:, :DC],
        W10[:, DC:], W11[:, DC:], W12[:, DC:],
    ])
    tabs = _make_tables(x, wstack).reshape(10 * N, DC)

    pad = EPAD - E
    srcs, dsts, vals = [], [], []
    for idx, val in ((s0_idx, s0_val), (s1_idx, s1_val), (s2_idx, s2_val)):
        idx = idx.astype(jnp.int32)
        srcs.append(jnp.pad(idx[1], (0, pad)))
        dsts.append(jnp.pad(idx[0], (0, pad)))
        vals.append(jnp.pad(val.astype(jnp.float32), (0, pad)))

    srcw = jnp.concatenate([srcs[TMAP[t]] + t * N for t in range(10)])
    dstw = jnp.concatenate([dsts[TMAP[t]] for t in range(10)])
    valw = jnp.concatenate([vals[TMAP[t]] for t in range(10)])

    zer = jnp.zeros((RPT, DC), jnp.float32)
    part = _sc_spmm(tabs, srcw, dstw, valw, zer)
    return _epilogue(part, x)


# speedup vs baseline: 1.5218x; 1.5218x over previous
import functools

import jax
import jax.numpy as jnp
from jax import lax
from jax.experimental import pallas as pl
from jax.experimental.pallas import tpu as pltpu
from jax.experimental.pallas import tpu_sc as plsc

N = 10000
NPAD = 10112
RPT = 632
D = 256
DC = 128
E = 160000
NC = 2
NS = 16
B = 128
EPAD = 163840
NBTOT = EPAD // B
NB0 = 72
NB1 = 8

CHUNK_T0 = [0, 2, 4, 7]
CHUNK_NS = [2, 2, 3, 3]
TMAP = [0, 1, 0, 1, 0, 1, 2, 0, 1, 2]


def _mm_body(x_ref, w_ref, o_ref):
    o_ref[0] = jnp.dot(x_ref[...], w_ref[0], preferred_element_type=jnp.float32)


def _make_tables(x, wstack):
    BN = 2000
    return pl.pallas_call(
        _mm_body,
        grid=(N // BN, 10),
        in_specs=[
            pl.BlockSpec((BN, D), lambda i, t: (i, 0)),
            pl.BlockSpec((1, D, DC), lambda i, t: (t, 0, 0)),
        ],
        out_specs=pl.BlockSpec((1, BN, DC), lambda i, t: (t, i, 0)),
        out_shape=jax.ShapeDtypeStruct((10, N, DC), jnp.float32),
    )(x, wstack)


def _sc_body(tabs, srcw, dstw, valw, zer,
             out, acc, sb3, db3, vb3, rows2, gsem, ssem, esem):
    cid = lax.axis_index("c")
    sid = lax.axis_index("s")
    nb = jnp.where(cid == 0, NB0, NB1)
    ebase = jnp.where(cid == 0, sid * NB0, NS * NB0 + sid * NB1) * B

    for c in range(4):
        pltpu.sync_copy(zer, acc.at[pl.ds(sid * RPT, RPT), :])
        plsc.subcore_barrier()

        def _subtask(j, _, t0=CHUNK_T0[c]):
            tb = (t0 + j) * EPAD + ebase

            def _esmall(b, s3):
                off = tb + b * B
                pltpu.async_copy(srcw.at[pl.ds(off, B)], sb3.at[s3],
                                 esem.at[s3])
                pltpu.async_copy(dstw.at[pl.ds(off, B)], db3.at[s3],
                                 esem.at[s3])
                pltpu.async_copy(valw.at[pl.ds(off, B)], vb3.at[s3],
                                 esem.at[s3])

            def _ewait(b, s3):
                off = tb + b * B
                pltpu.make_async_copy(srcw.at[pl.ds(off, B)], sb3.at[s3],
                                      esem.at[s3]).wait()
                pltpu.make_async_copy(dstw.at[pl.ds(off, B)], db3.at[s3],
                                      esem.at[s3]).wait()
                pltpu.make_async_copy(valw.at[pl.ds(off, B)], vb3.at[s3],
                                      esem.at[s3]).wait()

            def _gather(s3, bi):
                pltpu.async_copy(tabs.at[sb3.at[s3]], rows2.at[bi],
                                 gsem.at[bi])

            def _gwait(s3, bi):
                pltpu.make_async_copy(tabs.at[sb3.at[s3]], rows2.at[bi],
                                      gsem.at[bi]).wait()

            def _swait(s3, bi):
                pltpu.make_async_copy(rows2.at[bi], acc.at[db3.at[s3]],
                                      ssem.at[bi]).wait()

            _esmall(0, 0)
            _ewait(0, 0)
            _gather(0, 0)
            _esmall(1, 1)

            def _batch(b, _):
                bi = b % 2
                bo = (b + 1) % 2
                s3 = b % 3

                @pl.when(b + 1 < nb)
                def _():
                    @pl.when(b >= 1)
                    def _():
                        _swait((b - 1) % 3, bo)
                    _ewait(b + 1, (b + 1) % 3)
                    _gather((b + 1) % 3, bo)

                _gwait(s3, bi)

                @plsc.parallel_loop(0, 8, unroll=2)
                def _scale(g):
                    vv = vb3[s3, pl.ds(g * 16, 16)]
                    for l in range(16):
                        v = vv[l]
                        e = g * 16 + l
                        loads = [rows2[bi, e, pl.ds(jj * 16, 16)]
                                 for jj in range(8)]
                        for jj in range(8):
                            rows2[bi, e, pl.ds(jj * 16, 16)] = loads[jj] * v

                pltpu.async_copy(rows2.at[bi], acc.at[db3.at[s3]],
                                 ssem.at[bi], add=True)

                @pl.when(b + 2 < nb)
                def _():
                    _esmall(b + 2, (b + 2) % 3)
                return 0

            lax.fori_loop(0, nb, _batch, 0)
            _swait((nb - 2) % 3, nb % 2)
            _swait((nb - 1) % 3, (nb - 1) % 2)
            return 0

        lax.fori_loop(0, CHUNK_NS[c], _subtask, 0)

        plsc.subcore_barrier()
        pltpu.sync_copy(acc.at[pl.ds(sid * RPT, RPT), :],
                        out.at[cid, c, pl.ds(sid * RPT, RPT), :])


def _sc_spmm(tabs, srcw, dstw, valw, zer):
    mesh = plsc.VectorSubcoreMesh(core_axis_name="c", subcore_axis_name="s")
    kern = functools.partial(
        pl.kernel,
        mesh=mesh,
        out_type=jax.ShapeDtypeStruct((NC, 4, NPAD, DC), jnp.float32),
        scratch_types=[
            pltpu.VMEM_SHARED((NPAD, DC), jnp.float32),
            pltpu.VMEM((3, B), jnp.int32),
            pltpu.VMEM((3, B), jnp.int32),
            pltpu.VMEM((3, B), jnp.float32),
            pltpu.VMEM((2, B, DC), jnp.float32),
            pltpu.SemaphoreType.DMA((2,)),
            pltpu.SemaphoreType.DMA((2,)),
            pltpu.SemaphoreType.DMA((3,)),
        ],
    )(_sc_body)
    return kern(tabs, srcw, dstw, valw, zer)


def _ep_body(p_ref, x_ref, o_ref):
    c0 = p_ref[0, 0] + p_ref[1, 0]
    c1 = p_ref[0, 1] + p_ref[1, 1]
    c2 = p_ref[0, 2] + p_ref[1, 2]
    c3 = p_ref[0, 3] + p_ref[1, 3]
    o_ref[:, 0:DC] = jnp.maximum(jnp.maximum(c0, c2), 0.0)
    o_ref[:, DC:2 * DC] = jnp.maximum(jnp.maximum(c1, c3), 0.0)
    o_ref[:, 2 * DC:] = x_ref[...]


def _epilogue(part, x):
    BN = 2000
    return pl.pallas_call(
        _ep_body,
        grid=(N // BN,),
        in_specs=[
            pl.BlockSpec((NC, 4, BN, DC), lambda i: (0, 0, i, 0)),
            pl.BlockSpec((BN, D), lambda i: (i, 0)),
        ],
        out_specs=pl.BlockSpec((BN, 2 * D), lambda i: (i, 0)),
        out_shape=jax.ShapeDtypeStruct((N, 2 * D), jnp.float32),
    )(part, x)


def kernel(x, s0_idx, s0_val, s1_idx, s1_val, s2_idx, s2_val,
           W00, W01, W10, W11, W12):
    x = x.astype(jnp.float32)
    wstack = jnp.stack([
        W00[:, :DC], W01[:, :DC],
        W00[:, DC:], W01[:, DC:],
        W10[:, :DC], W11[:, :DC], W12[:, :DC],
        W10[:, DC:], W11[:, DC:], W12[:, DC:],
    ])
    tabs = _make_tables(x, wstack).reshape(10 * N, DC)

    pad = EPAD - E
    srcs, dsts, vals = [], [], []
    for idx, val in ((s0_idx, s0_val), (s1_idx, s1_val), (s2_idx, s2_val)):
        idx = idx.astype(jnp.int32)
        srcs.append(jnp.pad(idx[1], (0, pad)))
        dsts.append(jnp.pad(idx[0], (0, pad)))
        vals.append(jnp.pad(val.astype(jnp.float32), (0, pad)))

    srcw = jnp.concatenate([srcs[TMAP[t]] + t * N for t in range(10)])
    dstw = jnp.concatenate([dsts[TMAP[t]] for t in range(10)])
    valw = jnp.concatenate([vals[TMAP[t]] for t in range(10)])

    zer = jnp.zeros((RPT, DC), jnp.float32)
    part = _sc_spmm(tabs, srcw, dstw, valw, zer)
    return _epilogue(part, x)
